# baseline (speedup 1.0000x reference)
import functools

import jax
import jax.numpy as jnp
from jax.experimental import pallas as pl
from jax.experimental.pallas import tpu as pltpu

T, A, D, H, E = 8, 2, 1024, 2048, 8
HB = 2048
NH = H // HB


def _ffn_body(m_ref, x_ref, ei_ref, wg_ref, wu_ref, wd_ref, out_ref):
    h = pl.program_id(0)
    e = pl.program_id(1)

    @pl.when((e == 0) & (h == 0))
    def _init():
        out_ref[...] = jnp.zeros_like(out_ref)

    first = jnp.logical_or(e == 0, m_ref[e] != m_ref[jnp.maximum(e - 1, 0)])

    @pl.when(first)
    def _compute():
        mask = ei_ref[...] == m_ref[e]
        xb = x_ref[...]
        dn = (((1,), (1,)), ((), ()))
        g = jax.lax.dot_general(xb, wg_ref[0], dn,
                                preferred_element_type=jnp.float32)
        u = jax.lax.dot_general(xb, wu_ref[0], dn,
                                preferred_element_type=jnp.float32)
        act = (g * jax.lax.logistic(g)) * u
        y = jax.lax.dot_general(act, wd_ref[0], dn,
                                preferred_element_type=jnp.float32)
        out_ref[...] += jnp.where(mask, y, 0.0)


@jax.jit
def kernel(x, expert_indices, w_gate, w_up, w_down):
    x2 = jnp.repeat(x, A, axis=0)
    ei_flat = expert_indices.reshape(T * A).astype(jnp.int32)
    ei2 = ei_flat.reshape(T * A, 1)
    emap = jnp.sort(ei_flat)

    grid = (NH, T * A)
    out = pl.pallas_call(
        _ffn_body,
        grid_spec=pltpu.PrefetchScalarGridSpec(
            num_scalar_prefetch=1,
            grid=grid,
            in_specs=[
                pl.BlockSpec((T * A, D), lambda h, e, m: (0, 0)),
                pl.BlockSpec((T * A, 1), lambda h, e, m: (0, 0)),
                pl.BlockSpec((1, HB, D), lambda h, e, m: (m[e], h, 0)),
                pl.BlockSpec((1, HB, D), lambda h, e, m: (m[e], h, 0)),
                pl.BlockSpec((1, D, HB), lambda h, e, m: (m[e], 0, h)),
            ],
            out_specs=pl.BlockSpec((T * A, D), lambda h, e, m: (0, 0)),
        ),
        out_shape=jax.ShapeDtypeStruct((T * A, D), jnp.float32),
    )(emap, x2, ei2, w_gate, w_up, w_down)
    return out.reshape(T, A, D)


# speedup vs baseline: 1.2792x; 1.2792x over previous
import jax
import jax.numpy as jnp
from jax.experimental import pallas as pl
from jax.experimental.pallas import tpu as pltpu

T, A, D, H, E = 8, 2, 1024, 2048, 8
HB = 2048
NH = H // HB


def _expert_meta(expert_indices):
    ids = jnp.arange(E, dtype=jnp.int32)
    used = jnp.any(expert_indices[:, :, None] == ids[None, None, :],
                   axis=(0, 1))
    rank = jnp.cumsum(used.astype(jnp.int32))
    count = rank[E - 1]
    match = (rank[None, :] == ids[:, None] + 1) & used[None, :]
    order = jnp.min(jnp.where(match, ids[None, :], jnp.int32(E)), axis=1)
    last = jnp.max(jnp.where(used, ids, jnp.int32(-1)))
    order = jnp.minimum(order, last)
    return jnp.concatenate([order, count[None]])


def _ffn_body(m_ref, x_ref, ei_ref, wg_ref, wu_ref, wd_ref, out_ref):
    h = pl.program_id(0)
    e = pl.program_id(1)

    @pl.when((e == 0) & (h == 0))
    def _init():
        out_ref[...] = jnp.zeros_like(out_ref)

    @pl.when(e < m_ref[E])
    def _compute():
        xb = x_ref[...]
        dn = (((1,), (1,)), ((), ()))
        g = jax.lax.dot_general(xb, wg_ref[0], dn,
                                preferred_element_type=jnp.float32)
        u = jax.lax.dot_general(xb, wu_ref[0], dn,
                                preferred_element_type=jnp.float32)
        act = (g * jax.lax.logistic(g)) * u
        y = jax.lax.dot_general(act, wd_ref[0], dn,
                                preferred_element_type=jnp.float32)
        for a in range(A):
            mask = ei_ref[:, a:a + 1] == m_ref[e]
            out_ref[a * T:(a + 1) * T, :] += jnp.where(mask, y, 0.0)


@jax.jit
def kernel(x, expert_indices, w_gate, w_up, w_down):
    ei = expert_indices.astype(jnp.int32)
    meta = _expert_meta(ei)

    grid = (NH, E)
    out = pl.pallas_call(
        _ffn_body,
        grid_spec=pltpu.PrefetchScalarGridSpec(
            num_scalar_prefetch=1,
            grid=grid,
            in_specs=[
                pl.BlockSpec((T, D), lambda h, e, m: (0, 0)),
                pl.BlockSpec((T, A), lambda h, e, m: (0, 0)),
                pl.BlockSpec((1, HB, D), lambda h, e, m: (m[e], h, 0)),
                pl.BlockSpec((1, HB, D), lambda h, e, m: (m[e], h, 0)),
                pl.BlockSpec((1, D, HB), lambda h, e, m: (m[e], 0, h)),
            ],
            out_specs=pl.BlockSpec((A * T, D), lambda h, e, m: (0, 0)),
        ),
        out_shape=jax.ShapeDtypeStruct((A * T, D), jnp.float32),
    )(meta, x, ei, w_gate, w_up, w_down)
    return out.reshape(A, T, D).transpose(1, 0, 2)
